# baseline (speedup 1.0000x reference)
import functools

import jax
import jax.numpy as jnp
from jax import lax
from jax.experimental import pallas as pl
from jax.experimental.pallas import tpu as pltpu
from jax.experimental.pallas import tpu_sc as plsc


def _gather_kernel(N, D, chunk, nbuf):
    info = plsc.get_sparse_core_info()
    NC, NS = info.num_cores, info.num_subcores
    NW = NC * NS
    assert N % (NW * chunk) == 0
    n = N // (NW * chunk)
    assert n >= nbuf
    per_w = N // NW

    mesh = plsc.VectorSubcoreMesh(core_axis_name="c", subcore_axis_name="s")

    @functools.partial(
        pl.kernel,
        out_type=jax.ShapeDtypeStruct((N, D), jnp.float32),
        mesh=mesh,
        scratch_types=[
            [pltpu.VMEM((chunk,), jnp.int32) for _ in range(nbuf)],
            [pltpu.VMEM((chunk, D), jnp.float32) for _ in range(nbuf)],
            [pltpu.SemaphoreType.DMA for _ in range(nbuf)],
            [pltpu.SemaphoreType.DMA for _ in range(nbuf)],
        ],
        compiler_params=pltpu.CompilerParams(use_tc_tiling_on_sc=False),
    )
    def k(idx_hbm, table_hbm, out_hbm, idx_v, rows_v, sem_g, sem_s):
        wid = lax.axis_index("s") * NC + lax.axis_index("c")
        base = wid * per_w

        def stage_idx(c, b):
            pltpu.sync_copy(idx_hbm.at[pl.ds(base + c * chunk, chunk)], idx_v[b])

        def start_gather(b):
            pltpu.async_copy(table_hbm.at[idx_v[b]], rows_v[b], sem_g[b])

        def wait_gather(b):
            pltpu.make_async_copy(table_hbm.at[idx_v[b]], rows_v[b], sem_g[b]).wait()

        def store(c, b, wait):
            src = rows_v[b]
            dst = out_hbm.at[pl.ds(base + c * chunk, chunk)]
            if wait:
                pltpu.make_async_copy(src, dst, sem_s[b]).wait()
            else:
                pltpu.async_copy(src, dst, sem_s[b])

        for b in range(nbuf):
            stage_idx(b, b)
            start_gather(b)

        @pl.loop(0, n, step=nbuf)
        def _(g):
            for b in range(nbuf):
                c = g + b
                wait_gather(b)
                store(c, b, wait=False)

                @pl.when(c + nbuf < n)
                def _():
                    stage_idx(c + nbuf, b)

                store(c, b, wait=True)

                @pl.when(c + nbuf < n)
                def _():
                    start_gather(b)

    return k


def kernel(tokens, embedding_weight):
    B, T = tokens.shape
    V, D = embedding_weight.shape
    S = 4
    TS = T // S
    k = _gather_kernel(B * TS, D, chunk=800, nbuf=2)
    outs = []
    for s in range(S):
        flat = tokens[:, s * TS:(s + 1) * TS].reshape(B * TS).astype(jnp.int32)
        outs.append(k(flat, embedding_weight).reshape(B, TS, D))
    return jnp.concatenate(outs, axis=1)


# speedup vs baseline: 2.8837x; 2.8837x over previous
import functools

import jax
import jax.numpy as jnp
from jax import lax
from jax.experimental import pallas as pl
from jax.experimental.pallas import tpu as pltpu
from jax.experimental.pallas import tpu_sc as plsc


def _gather_kernel(N, D, chunk, nbuf):
    info = plsc.get_sparse_core_info()
    NC, NS = info.num_cores, info.num_subcores
    NW = NC * NS
    assert N % (NW * chunk) == 0
    n = N // (NW * chunk)
    assert n >= nbuf
    per_w = N // NW

    mesh = plsc.VectorSubcoreMesh(core_axis_name="c", subcore_axis_name="s")

    @functools.partial(
        pl.kernel,
        out_type=jax.ShapeDtypeStruct((N, D), jnp.float32),
        mesh=mesh,
        scratch_types=[
            [pltpu.VMEM((chunk,), jnp.int32) for _ in range(nbuf)],
            [pltpu.VMEM((chunk, D), jnp.float32) for _ in range(nbuf)],
            [pltpu.SemaphoreType.DMA for _ in range(nbuf)],
            [pltpu.SemaphoreType.DMA for _ in range(nbuf)],
        ],
        compiler_params=pltpu.CompilerParams(use_tc_tiling_on_sc=False),
    )
    def k(idx_hbm, table_hbm, out_hbm, idx_v, rows_v, sem_g, sem_s):
        wid = lax.axis_index("s") * NC + lax.axis_index("c")
        base = wid * per_w

        def stage_idx(c, b):
            pltpu.sync_copy(idx_hbm.at[pl.ds(base + c * chunk, chunk)], idx_v[b])

        def start_gather(b):
            pltpu.async_copy(table_hbm.at[idx_v[b]], rows_v[b], sem_g[b])

        def wait_gather(b):
            pltpu.make_async_copy(table_hbm.at[idx_v[b]], rows_v[b], sem_g[b]).wait()

        def store(c, b, wait):
            src = rows_v[b]
            dst = out_hbm.at[pl.ds(base + c * chunk, chunk)]
            if wait:
                pltpu.make_async_copy(src, dst, sem_s[b]).wait()
            else:
                pltpu.async_copy(src, dst, sem_s[b])

        for b in range(nbuf):
            stage_idx(b, b)
            start_gather(b)

        @pl.loop(0, n, step=nbuf)
        def _(g):
            for b in range(nbuf):
                c = g + b
                wait_gather(b)
                store(c, b, wait=False)

                @pl.when(c + nbuf < n)
                def _():
                    stage_idx(c + nbuf, b)

                store(c, b, wait=True)

                @pl.when(c + nbuf < n)
                def _():
                    start_gather(b)

    return k


def kernel(tokens, embedding_weight):
    B, T = tokens.shape
    V, D = embedding_weight.shape
    k = _gather_kernel(B * T, D, chunk=800, nbuf=2)
    flat_idx = tokens.reshape(B * T).astype(jnp.int32)
    out = k(flat_idx, embedding_weight)
    return out.reshape(B, T, D)
